# Initial kernel scaffold; baseline (speedup 1.0000x reference)
import jax
import jax.numpy as jnp
from jax.experimental import pallas as pl


def kernel(x, token_table, pos_table):
    raise NotImplementedError("write your pallas kernel here")


# speedup vs baseline: 1.3625x; 1.3625x over previous
import functools

import jax
import jax.numpy as jnp
from jax import lax
from jax.experimental import pallas as pl
from jax.experimental.pallas import tpu as pltpu
from jax.experimental.pallas import tpu_sc as plsc

_B = 4096
_L = 200
_D = 32
_NW = 32
_IDXW = 100
_SEQ_PER_CHUNK = 4
_CHUNK_IDX = _SEQ_PER_CHUNK * _L
_CHUNK_ROWS = _CHUNK_IDX // _IDXW
_TOT_IDX = _B * _L
_IDX_ROWS = _TOT_IDX // _IDXW
_ROWS_PER_W = _IDX_ROWS // _NW
_CHUNKS_PER_W = _ROWS_PER_W // _CHUNK_ROWS


def _build():
  mesh = plsc.VectorSubcoreMesh(core_axis_name="c", subcore_axis_name="s")

  @functools.partial(
      pl.kernel,
      mesh=mesh,
      compiler_params=pltpu.CompilerParams(use_tc_tiling_on_sc=False),
      out_type=jax.ShapeDtypeStruct((_TOT_IDX, _D), jnp.float32),
      scratch_types=[
          pltpu.VMEM((_CHUNK_ROWS, _IDXW), jnp.int32),
          pltpu.VMEM((_CHUNK_IDX, _D), jnp.float32),
          pltpu.VMEM((_L, _D), jnp.float32),
          pltpu.SemaphoreType.DMA,
      ],
  )
  def k(x_hbm, table_hbm, pos_hbm, out_hbm, idx_v, rows_v, pos_v, sem):
    wid = lax.axis_index("c") * 16 + lax.axis_index("s")
    pltpu.sync_copy(pos_hbm, pos_v)

    @pl.loop(0, _CHUNKS_PER_W)
    def _chunk(c):
      base_row = wid * _ROWS_PER_W + c * _CHUNK_ROWS
      pltpu.sync_copy(x_hbm.at[pl.ds(base_row, _CHUNK_ROWS)], idx_v)
      copies = [
          pltpu.async_copy(
              table_hbm.at[idx_v.at[j]],
              rows_v.at[pl.ds(j * _IDXW, _IDXW)],
              sem,
          )
          for j in range(_CHUNK_ROWS)
      ]
      for cp in copies:
        cp.wait()
      for h in range(2):
        @pl.loop(0, _L)
        def _row(r):
          p = pos_v[r, pl.ds(h * 16, 16)]
          for s in range(_SEQ_PER_CHUNK):
            rows_v[s * _L + r, pl.ds(h * 16, 16)] += p
      pltpu.sync_copy(
          rows_v, out_hbm.at[pl.ds(base_row * _IDXW, _CHUNK_IDX)])

  return k


_k = _build()


def kernel(x, token_table, pos_table):
  xf = x.reshape(_IDX_ROWS, _IDXW)
  out = _k(xf, token_table, pos_table)
  return out.reshape(_B, _L, _D)


# speedup vs baseline: 1.4855x; 1.0903x over previous
import functools

import jax
import jax.numpy as jnp
from jax import lax
from jax.experimental import pallas as pl
from jax.experimental.pallas import tpu as pltpu
from jax.experimental.pallas import tpu_sc as plsc

_B = 4096
_L = 200
_D = 32
_NW = 32
_IDXW = 100
_SEQ_PER_CHUNK = 4
_CHUNK_IDX = _SEQ_PER_CHUNK * _L
_CHUNK_ROWS = _CHUNK_IDX // _IDXW
_TOT_IDX = _B * _L
_IDX_ROWS = _TOT_IDX // _IDXW
_ROWS_PER_W = _IDX_ROWS // _NW
_CHUNKS_PER_W = _ROWS_PER_W // _CHUNK_ROWS
_NBODY = _CHUNKS_PER_W // 2


def _build():
  mesh = plsc.VectorSubcoreMesh(core_axis_name="c", subcore_axis_name="s")

  @functools.partial(
      pl.kernel,
      mesh=mesh,
      compiler_params=pltpu.CompilerParams(use_tc_tiling_on_sc=False),
      out_type=jax.ShapeDtypeStruct((_TOT_IDX, _D), jnp.float32),
      scratch_types=[
          pltpu.VMEM((_CHUNK_ROWS, _IDXW), jnp.int32),
          pltpu.VMEM((_CHUNK_ROWS, _IDXW), jnp.int32),
          pltpu.VMEM((_CHUNK_IDX, _D), jnp.float32),
          pltpu.VMEM((_CHUNK_IDX, _D), jnp.float32),
          pltpu.VMEM((_L, _D), jnp.float32),
          pltpu.SemaphoreType.DMA,
          pltpu.SemaphoreType.DMA,
          pltpu.SemaphoreType.DMA,
          pltpu.SemaphoreType.DMA,
          pltpu.SemaphoreType.DMA,
          pltpu.SemaphoreType.DMA,
      ],
  )
  def k(x_hbm, table_hbm, pos_hbm, out_hbm,
        idx0, idx1, rows0, rows1, pos_v,
        si0, si1, sg0, sg1, so0, so1):
    wid = lax.axis_index("c") * 16 + lax.axis_index("s")

    def start_idx(chunk, ibuf, sem):
      base = wid * _ROWS_PER_W + chunk * _CHUNK_ROWS
      pltpu.async_copy(x_hbm.at[pl.ds(base, _CHUNK_ROWS)], ibuf, sem)

    def drain_idx(ibuf, sem):
      pltpu.make_async_copy(
          x_hbm.at[pl.ds(0, _CHUNK_ROWS)], ibuf, sem).wait()

    def fire_gathers(ibuf, rbuf, sem):
      for j in range(_CHUNK_ROWS):
        pltpu.async_copy(
            table_hbm.at[ibuf.at[j]],
            rbuf.at[pl.ds(j * _IDXW, _IDXW)],
            sem,
        )

    def drain_gathers(rbuf, sem):
      pltpu.make_async_copy(
          table_hbm.at[pl.ds(0, _CHUNK_IDX)], rbuf, sem).wait()

    def start_out(chunk, rbuf, sem):
      base = (wid * _ROWS_PER_W + chunk * _CHUNK_ROWS) * _IDXW
      pltpu.async_copy(rbuf, out_hbm.at[pl.ds(base, _CHUNK_IDX)], sem)

    def drain_out(rbuf, sem):
      pltpu.make_async_copy(
          rbuf, out_hbm.at[pl.ds(0, _CHUNK_IDX)], sem).wait()

    def pos_add(rbuf):
      for h in range(2):
        @plsc.parallel_loop(0, _L, unroll=4)
        def _row(r):
          p = pos_v[r, pl.ds(h * 16, 16)]
          for s in range(_SEQ_PER_CHUNK):
            rbuf[s * _L + r, pl.ds(h * 16, 16)] += p

    pltpu.sync_copy(pos_hbm, pos_v)
    start_idx(0, idx0, si0)
    drain_idx(idx0, si0)
    fire_gathers(idx0, rows0, sg0)
    start_idx(1, idx1, si1)

    @pl.loop(0, _NBODY)
    def _body(i):
      c0 = i * 2

      drain_gathers(rows0, sg0)
      drain_idx(idx1, si1)

      @pl.when(i > 0)
      def _():
        drain_out(rows1, so1)

      fire_gathers(idx1, rows1, sg1)

      @pl.when(i < _NBODY - 1)
      def _():
        start_idx(c0 + 2, idx0, si0)

      pos_add(rows0)
      start_out(c0, rows0, so0)

      drain_gathers(rows1, sg1)

      @pl.when(i < _NBODY - 1)
      def _():
        drain_idx(idx0, si0)

      drain_out(rows0, so0)

      @pl.when(i < _NBODY - 1)
      def _():
        fire_gathers(idx0, rows0, sg0)
        start_idx(c0 + 3, idx1, si1)

      pos_add(rows1)
      start_out(c0 + 1, rows1, so1)

    drain_out(rows1, so1)

  return k


_k = _build()


def kernel(x, token_table, pos_table):
  xf = x.reshape(_IDX_ROWS, _IDXW)
  out = _k(xf, token_table, pos_table)
  return out.reshape(_B, _L, _D)
